# baseline (speedup 1.0000x reference)
import functools

import jax
import jax.numpy as jnp
from jax import lax
from jax.experimental import pallas as pl
from jax.experimental.pallas import tpu as pltpu
from jax.experimental.pallas import tpu_sc as plsc

D = 768
E = 64
H = 2048
TR = 32
T = 2048
TP = 4096


TB = 128


def _router_body(x_ref, wr_ref, p_ref, nts_ref, ps_ref):
    logits = lax.dot_general(
        x_ref[...], wr_ref[...], (((1,), (1,)), ((), ())),
        preferred_element_type=jnp.float32)
    m = jnp.max(logits, axis=1, keepdims=True)
    col = lax.broadcasted_iota(jnp.int32, logits.shape, 1)
    eid = jnp.min(jnp.where(logits >= m, col, E), axis=1)
    oh = (col == eid[:, None]).astype(jnp.float32)

    r_i = lax.broadcasted_iota(jnp.int32, (TB, TB), 0)
    c_i = lax.broadcasted_iota(jnp.int32, (TB, TB), 1)
    Ltri = (c_i <= r_i).astype(jnp.float32)
    carry = jnp.zeros((1, E), jnp.float32)
    blocks = []
    for b in range(T // TB):
        blk = oh[b * TB:(b + 1) * TB, :]
        Cb = lax.dot_general(Ltri, blk, (((1,), (0,)), ((), ())),
                             preferred_element_type=jnp.float32) + carry
        blocks.append(Cb)
        carry = Cb[TB - 1:TB, :]
    C = jnp.concatenate(blocks, axis=0)
    rank = jnp.sum(C * oh, axis=1) - 1.0

    counts = carry
    ntsf = jnp.floor((counts + (TR - 1)) * (1.0 / TR))
    re_ = lax.broadcasted_iota(jnp.int32, (E, E), 0)
    ce_ = lax.broadcasted_iota(jnp.int32, (E, E), 1)
    Lex = (re_ < ce_).astype(jnp.float32)
    psf = lax.dot_general(ntsf, Lex, (((1,), (0,)), ((), ())),
                          preferred_element_type=jnp.float32) * TR
    pstart_tok = jnp.sum(oh * psf, axis=1)
    p_ref[...] = (pstart_tok + rank).astype(jnp.int32)
    nts_ref[...] = ntsf[0].astype(jnp.int32)
    ps_ref[...] = psf[0].astype(jnp.int32)


def _router(x_flat, Wr):
    return pl.pallas_call(
        _router_body,
        out_shape=(jax.ShapeDtypeStruct((T,), jnp.int32),
                   jax.ShapeDtypeStruct((E,), jnp.int32),
                   jax.ShapeDtypeStruct((E,), jnp.int32)),
    )(x_flat, Wr)


def _make_sc_gather(Dm, B):
    info = plsc.get_sparse_core_info()
    NC, NS = info.num_cores, info.num_subcores
    NW = NC * NS
    assert B % (8 * NW) == 0
    b_per_w = B // NW
    CH = min(b_per_w, 64)
    n_ch = b_per_w // CH
    assert b_per_w % CH == 0
    mesh = plsc.VectorSubcoreMesh(core_axis_name="c", subcore_axis_name="s")

    @functools.partial(
        pl.kernel, mesh=mesh,
        out_type=jax.ShapeDtypeStruct((B, Dm), jnp.float32),
        scratch_types=[
            pltpu.VMEM((CH,), jnp.int32),
            pltpu.VMEM((CH, Dm), jnp.float32),
            pltpu.SemaphoreType.DMA,
        ],
    )
    def k(table_hbm, idx_hbm, out_hbm, idx_v, rows_v, sem):
        wid = lax.axis_index("s") * NC + lax.axis_index("c")
        base = wid * b_per_w
        for c in range(n_ch):
            off = base + c * CH
            pltpu.sync_copy(idx_hbm.at[pl.ds(off, CH)], idx_v)
            pltpu.async_copy(table_hbm.at[idx_v], rows_v, sem).wait()
            pltpu.sync_copy(rows_v, out_hbm.at[pl.ds(off, CH)])

    return k


def _make_sc_scatter(Dm, N, B):
    info = plsc.get_sparse_core_info()
    NC, NS = info.num_cores, info.num_subcores
    NW = NC * NS
    assert N % (8 * NW) == 0
    n_per_w = N // NW
    CH = min(n_per_w, 64)
    n_ch = n_per_w // CH
    assert n_per_w % CH == 0
    mesh = plsc.VectorSubcoreMesh(core_axis_name="c", subcore_axis_name="s")

    @functools.partial(
        pl.kernel, mesh=mesh,
        out_type=jax.ShapeDtypeStruct((B, Dm), jnp.float32),
        scratch_types=[
            pltpu.VMEM((CH,), jnp.int32),
            pltpu.VMEM((CH, Dm), jnp.float32),
            pltpu.SemaphoreType.DMA,
        ],
    )
    def k(x_hbm, p_hbm, out_hbm, idx_v, rows_v, sem):
        wid = lax.axis_index("s") * NC + lax.axis_index("c")
        base = wid * n_per_w
        for c in range(n_ch):
            off = base + c * CH
            pltpu.sync_copy(p_hbm.at[pl.ds(off, CH)], idx_v)
            pltpu.sync_copy(x_hbm.at[pl.ds(off, CH)], rows_v)
            pltpu.async_copy(rows_v, out_hbm.at[idx_v], sem).wait()

    return k


CHH = 512
NCH = H // CHH


def _mlp_body(ps_ref, nt_ref, x_ref, w12_ref, w3_ref, y_hbm,
              y_bufs, sem_out, cnt_ref):
    e = pl.program_id(0)
    ps = ps_ref[e]
    nt = nt_ref[e]

    @pl.when(e == 0)
    def _init():
        cnt_ref[0] = 0

    def tile(i, carry):
        r0 = pl.multiple_of(ps + i * TR, TR)
        xs = x_ref[pl.ds(r0, TR), :]
        acc = jnp.zeros((TR, D), jnp.float32)
        for c in range(NCH):
            w1c = w12_ref[0, c * CHH:(c + 1) * CHH, :]
            w2c = w12_ref[0, H + c * CHH:H + (c + 1) * CHH, :]
            h1 = lax.dot_general(xs, w1c, (((1,), (1,)), ((), ())),
                                 preferred_element_type=jnp.float32,
                                 precision=None)
            h2 = lax.dot_general(xs, w2c, (((1,), (1,)), ((), ())),
                                 preferred_element_type=jnp.float32,
                                 precision=None)
            g = (h1 / (1.0 + jnp.exp(-h1))) * h2
            w3c = w3_ref[0, :, c * CHH:(c + 1) * CHH]
            acc = acc + lax.dot_general(
                g, w3c, (((1,), (1,)), ((), ())),
                preferred_element_type=jnp.float32,
                precision=None)
        cnt = cnt_ref[0]
        k = lax.rem(cnt, 2)

        @pl.when(cnt >= 2)
        def _drain_prev():
            pltpu.make_async_copy(
                y_bufs.at[k], y_hbm.at[pl.ds(r0, TR), :], sem_out.at[k]
            ).wait()

        y_bufs[k] = acc
        pltpu.make_async_copy(
            y_bufs.at[k], y_hbm.at[pl.ds(r0, TR), :], sem_out.at[k]
        ).start()
        cnt_ref[0] = cnt + 1
        return carry

    lax.fori_loop(0, nt, tile, 0)

    @pl.when(e == E - 1)
    def _final_drain():
        cnt = cnt_ref[0]

        @pl.when(cnt >= 1)
        def _():
            k = lax.rem(cnt - 1, 2)
            pltpu.make_async_copy(
                y_bufs.at[k], y_hbm.at[pl.ds(0, TR), :], sem_out.at[k]
            ).wait()

        @pl.when(cnt >= 2)
        def _():
            k = lax.rem(cnt - 2, 2)
            pltpu.make_async_copy(
                y_bufs.at[k], y_hbm.at[pl.ds(0, TR), :], sem_out.at[k]
            ).wait()


def _mlp(pstarts, nts, x_pad, W12, W3):
    return pl.pallas_call(
        _mlp_body,
        grid=(E,),
        in_specs=[
            pl.BlockSpec(memory_space=pltpu.SMEM),
            pl.BlockSpec(memory_space=pltpu.SMEM),
            pl.BlockSpec((TP, D), lambda e: (0, 0)),
            pl.BlockSpec((1, 2 * H, D), lambda e: (e, 0, 0)),
            pl.BlockSpec((1, D, H), lambda e: (e, 0, 0)),
        ],
        out_specs=pl.BlockSpec(memory_space=pl.ANY),
        out_shape=jax.ShapeDtypeStruct((TP, D), jnp.float32),
        scratch_shapes=[
            pltpu.VMEM((2, TR, D), jnp.float32),
            pltpu.SemaphoreType.DMA((2,)),
            pltpu.SMEM((1,), jnp.int32),
        ],
    )(pstarts, nts, x_pad, W12, W3)


def kernel(x, Wr, W12, W3):
    B, S, Dm = x.shape
    x_flat = x.reshape(-1, Dm)

    p, nts, pstarts = _router(x_flat, Wr)

    x_pad = _make_sc_scatter(D, T, TP)(x_flat, p)
    y_pad = _mlp(pstarts, nts, x_pad, W12, W3)
    out = _make_sc_gather(D, T)(y_pad, p)
    return out.reshape(B, S, Dm)


# speedup vs baseline: 1.0928x; 1.0928x over previous
import functools

import jax
import jax.numpy as jnp
from jax import lax
from jax.experimental import pallas as pl
from jax.experimental.pallas import tpu as pltpu
from jax.experimental.pallas import tpu_sc as plsc

D = 768
E = 64
H = 2048
TR = 64
T = 2048
TP = 6144


TB = 128


def _router_body(x_ref, wr_ref, p_ref, nts_ref, ps_ref):
    logits = lax.dot_general(
        x_ref[...], wr_ref[...], (((1,), (1,)), ((), ())),
        preferred_element_type=jnp.float32)
    m = jnp.max(logits, axis=1, keepdims=True)
    col = lax.broadcasted_iota(jnp.int32, logits.shape, 1)
    eid = jnp.min(jnp.where(logits >= m, col, E), axis=1)
    oh = (col == eid[:, None]).astype(jnp.float32)

    r_i = lax.broadcasted_iota(jnp.int32, (TB, TB), 0)
    c_i = lax.broadcasted_iota(jnp.int32, (TB, TB), 1)
    Ltri = (c_i <= r_i).astype(jnp.float32)
    carry = jnp.zeros((1, E), jnp.float32)
    blocks = []
    for b in range(T // TB):
        blk = oh[b * TB:(b + 1) * TB, :]
        Cb = lax.dot_general(Ltri, blk, (((1,), (0,)), ((), ())),
                             preferred_element_type=jnp.float32) + carry
        blocks.append(Cb)
        carry = Cb[TB - 1:TB, :]
    C = jnp.concatenate(blocks, axis=0)
    rank = jnp.sum(C * oh, axis=1) - 1.0

    counts = carry
    ntsf = jnp.floor((counts + (TR - 1)) * (1.0 / TR))
    re_ = lax.broadcasted_iota(jnp.int32, (E, E), 0)
    ce_ = lax.broadcasted_iota(jnp.int32, (E, E), 1)
    Lex = (re_ < ce_).astype(jnp.float32)
    psf = lax.dot_general(ntsf, Lex, (((1,), (0,)), ((), ())),
                          preferred_element_type=jnp.float32) * TR
    pstart_tok = jnp.sum(oh * psf, axis=1)
    p_ref[...] = (pstart_tok + rank).astype(jnp.int32)
    nts_ref[...] = ntsf[0].astype(jnp.int32)
    ps_ref[...] = psf[0].astype(jnp.int32)


def _router(x_flat, Wr):
    return pl.pallas_call(
        _router_body,
        out_shape=(jax.ShapeDtypeStruct((T,), jnp.int32),
                   jax.ShapeDtypeStruct((E,), jnp.int32),
                   jax.ShapeDtypeStruct((E,), jnp.int32)),
    )(x_flat, Wr)


def _make_sc_gather(Dm, B):
    info = plsc.get_sparse_core_info()
    NC, NS = info.num_cores, info.num_subcores
    NW = NC * NS
    assert B % (8 * NW) == 0
    b_per_w = B // NW
    CH = min(b_per_w, 64)
    n_ch = b_per_w // CH
    assert b_per_w % CH == 0
    mesh = plsc.VectorSubcoreMesh(core_axis_name="c", subcore_axis_name="s")

    @functools.partial(
        pl.kernel, mesh=mesh,
        out_type=jax.ShapeDtypeStruct((B, Dm), jnp.float32),
        scratch_types=[
            pltpu.VMEM((CH,), jnp.int32),
            pltpu.VMEM((CH, Dm), jnp.float32),
            pltpu.SemaphoreType.DMA,
        ],
    )
    def k(table_hbm, idx_hbm, out_hbm, idx_v, rows_v, sem):
        wid = lax.axis_index("s") * NC + lax.axis_index("c")
        base = wid * b_per_w
        for c in range(n_ch):
            off = base + c * CH
            pltpu.sync_copy(idx_hbm.at[pl.ds(off, CH)], idx_v)
            pltpu.async_copy(table_hbm.at[idx_v], rows_v, sem).wait()
            pltpu.sync_copy(rows_v, out_hbm.at[pl.ds(off, CH)])

    return k


def _make_sc_scatter(Dm, N, B):
    info = plsc.get_sparse_core_info()
    NC, NS = info.num_cores, info.num_subcores
    NW = NC * NS
    assert N % (8 * NW) == 0
    n_per_w = N // NW
    CH = min(n_per_w, 64)
    n_ch = n_per_w // CH
    assert n_per_w % CH == 0
    mesh = plsc.VectorSubcoreMesh(core_axis_name="c", subcore_axis_name="s")

    @functools.partial(
        pl.kernel, mesh=mesh,
        out_type=jax.ShapeDtypeStruct((B, Dm), jnp.float32),
        scratch_types=[
            pltpu.VMEM((CH,), jnp.int32),
            pltpu.VMEM((CH, Dm), jnp.float32),
            pltpu.SemaphoreType.DMA,
        ],
    )
    def k(x_hbm, p_hbm, out_hbm, idx_v, rows_v, sem):
        wid = lax.axis_index("s") * NC + lax.axis_index("c")
        base = wid * n_per_w
        for c in range(n_ch):
            off = base + c * CH
            pltpu.sync_copy(p_hbm.at[pl.ds(off, CH)], idx_v)
            pltpu.sync_copy(x_hbm.at[pl.ds(off, CH)], rows_v)
            pltpu.async_copy(rows_v, out_hbm.at[idx_v], sem).wait()

    return k


CHH = 512
NCH = H // CHH


def _mlp_body(ps_ref, nt_ref, x_ref, w12_ref, w3_ref, y_hbm,
              y_bufs, sem_out, cnt_ref):
    e = pl.program_id(0)
    ps = ps_ref[e]
    nt = nt_ref[e]

    @pl.when(e == 0)
    def _init():
        cnt_ref[0] = 0

    def tile(i, carry):
        r0 = pl.multiple_of(ps + i * TR, TR)
        xs = x_ref[pl.ds(r0, TR), :]
        acc = jnp.zeros((TR, D), jnp.float32)
        for c in range(NCH):
            w1c = w12_ref[0, c * CHH:(c + 1) * CHH, :]
            w2c = w12_ref[0, H + c * CHH:H + (c + 1) * CHH, :]
            h1 = lax.dot_general(xs, w1c, (((1,), (1,)), ((), ())),
                                 preferred_element_type=jnp.float32,
                                 precision=None)
            h2 = lax.dot_general(xs, w2c, (((1,), (1,)), ((), ())),
                                 preferred_element_type=jnp.float32,
                                 precision=None)
            g = (h1 / (1.0 + jnp.exp(-h1))) * h2
            w3c = w3_ref[0, :, c * CHH:(c + 1) * CHH]
            acc = acc + lax.dot_general(
                g, w3c, (((1,), (1,)), ((), ())),
                preferred_element_type=jnp.float32,
                precision=None)
        cnt = cnt_ref[0]
        k = lax.rem(cnt, 2)

        @pl.when(cnt >= 2)
        def _drain_prev():
            pltpu.make_async_copy(
                y_bufs.at[k], y_hbm.at[pl.ds(r0, TR), :], sem_out.at[k]
            ).wait()

        y_bufs[k] = acc
        pltpu.make_async_copy(
            y_bufs.at[k], y_hbm.at[pl.ds(r0, TR), :], sem_out.at[k]
        ).start()
        cnt_ref[0] = cnt + 1
        return carry

    lax.fori_loop(0, nt, tile, 0)

    @pl.when(e == E - 1)
    def _final_drain():
        cnt = cnt_ref[0]

        @pl.when(cnt >= 1)
        def _():
            k = lax.rem(cnt - 1, 2)
            pltpu.make_async_copy(
                y_bufs.at[k], y_hbm.at[pl.ds(0, TR), :], sem_out.at[k]
            ).wait()

        @pl.when(cnt >= 2)
        def _():
            k = lax.rem(cnt - 2, 2)
            pltpu.make_async_copy(
                y_bufs.at[k], y_hbm.at[pl.ds(0, TR), :], sem_out.at[k]
            ).wait()


def _mlp(pstarts, nts, x_pad, W12, W3):
    return pl.pallas_call(
        _mlp_body,
        grid=(E,),
        in_specs=[
            pl.BlockSpec(memory_space=pltpu.SMEM),
            pl.BlockSpec(memory_space=pltpu.SMEM),
            pl.BlockSpec((TP, D), lambda e: (0, 0)),
            pl.BlockSpec((1, 2 * H, D), lambda e: (e, 0, 0)),
            pl.BlockSpec((1, D, H), lambda e: (e, 0, 0)),
        ],
        out_specs=pl.BlockSpec(memory_space=pl.ANY),
        out_shape=jax.ShapeDtypeStruct((TP, D), jnp.float32),
        scratch_shapes=[
            pltpu.VMEM((2, TR, D), jnp.float32),
            pltpu.SemaphoreType.DMA((2,)),
            pltpu.SMEM((1,), jnp.int32),
        ],
    )(pstarts, nts, x_pad, W12, W3)


def kernel(x, Wr, W12, W3):
    B, S, Dm = x.shape
    x_flat = x.reshape(-1, Dm)

    p, nts, pstarts = _router(x_flat, Wr)

    x_pad = _make_sc_scatter(D, T, TP)(x_flat, p)
    y_pad = _mlp(pstarts, nts, x_pad, W12, W3)
    out = _make_sc_gather(D, T)(y_pad, p)
    return out.reshape(B, S, Dm)


# speedup vs baseline: 1.0946x; 1.0017x over previous
import functools

import jax
import jax.numpy as jnp
from jax import lax
from jax.experimental import pallas as pl
from jax.experimental.pallas import tpu as pltpu
from jax.experimental.pallas import tpu_sc as plsc

D = 768
E = 64
H = 2048
TR = 64
T = 2048
TP = 6144


TB = 128


def _router_body(x_ref, wr_ref, p_ref, nts_ref, ps_ref):
    logits = lax.dot_general(
        x_ref[...], wr_ref[...], (((1,), (1,)), ((), ())),
        preferred_element_type=jnp.float32)
    m = jnp.max(logits, axis=1, keepdims=True)
    col = lax.broadcasted_iota(jnp.int32, logits.shape, 1)
    eid = jnp.min(jnp.where(logits >= m, col, E), axis=1)
    oh = (col == eid[:, None]).astype(jnp.float32)

    r_i = lax.broadcasted_iota(jnp.int32, (TB, TB), 0)
    c_i = lax.broadcasted_iota(jnp.int32, (TB, TB), 1)
    Ltri = (c_i <= r_i).astype(jnp.float32)
    carry = jnp.zeros((1, E), jnp.float32)
    blocks = []
    for b in range(T // TB):
        blk = oh[b * TB:(b + 1) * TB, :]
        Cb = lax.dot_general(Ltri, blk, (((1,), (0,)), ((), ())),
                             preferred_element_type=jnp.float32) + carry
        blocks.append(Cb)
        carry = Cb[TB - 1:TB, :]
    C = jnp.concatenate(blocks, axis=0)
    rank = jnp.sum(C * oh, axis=1) - 1.0

    counts = carry
    ntsf = jnp.floor((counts + (TR - 1)) * (1.0 / TR))
    re_ = lax.broadcasted_iota(jnp.int32, (E, E), 0)
    ce_ = lax.broadcasted_iota(jnp.int32, (E, E), 1)
    Lex = (re_ < ce_).astype(jnp.float32)
    psf = lax.dot_general(ntsf, Lex, (((1,), (0,)), ((), ())),
                          preferred_element_type=jnp.float32) * TR
    pstart_tok = jnp.sum(oh * psf, axis=1)
    p_ref[...] = (pstart_tok + rank).astype(jnp.int32)
    nts_ref[...] = ntsf[0].astype(jnp.int32)
    ps_ref[...] = psf[0].astype(jnp.int32)


def _router(x_flat, Wr):
    return pl.pallas_call(
        _router_body,
        out_shape=(jax.ShapeDtypeStruct((T,), jnp.int32),
                   jax.ShapeDtypeStruct((E,), jnp.int32),
                   jax.ShapeDtypeStruct((E,), jnp.int32)),
    )(x_flat, Wr)


def _make_sc_gather(Dm, B):
    info = plsc.get_sparse_core_info()
    NC, NS = info.num_cores, info.num_subcores
    NW = NC * NS
    assert B % (8 * NW) == 0
    b_per_w = B // NW
    CH = min(b_per_w, 64)
    n_ch = b_per_w // CH
    assert b_per_w % CH == 0
    mesh = plsc.VectorSubcoreMesh(core_axis_name="c", subcore_axis_name="s")

    @functools.partial(
        pl.kernel, mesh=mesh,
        out_type=jax.ShapeDtypeStruct((B, Dm), jnp.float32),
        scratch_types=[
            pltpu.VMEM((CH,), jnp.int32),
            pltpu.VMEM((CH, Dm), jnp.float32),
            pltpu.SemaphoreType.DMA,
        ],
    )
    def k(table_hbm, idx_hbm, out_hbm, idx_v, rows_v, sem):
        wid = lax.axis_index("s") * NC + lax.axis_index("c")
        base = wid * b_per_w
        for c in range(n_ch):
            off = base + c * CH
            pltpu.sync_copy(idx_hbm.at[pl.ds(off, CH)], idx_v)
            pltpu.async_copy(table_hbm.at[idx_v], rows_v, sem).wait()
            pltpu.sync_copy(rows_v, out_hbm.at[pl.ds(off, CH)])

    return k


def _make_sc_scatter(Dm, N, B):
    info = plsc.get_sparse_core_info()
    NC, NS = info.num_cores, info.num_subcores
    NW = NC * NS
    assert N % (8 * NW) == 0
    n_per_w = N // NW
    CH = min(n_per_w, 64)
    n_ch = n_per_w // CH
    assert n_per_w % CH == 0
    mesh = plsc.VectorSubcoreMesh(core_axis_name="c", subcore_axis_name="s")

    @functools.partial(
        pl.kernel, mesh=mesh,
        out_type=jax.ShapeDtypeStruct((B, Dm), jnp.float32),
        scratch_types=[
            pltpu.VMEM((CH,), jnp.int32),
            pltpu.VMEM((CH, Dm), jnp.float32),
            pltpu.SemaphoreType.DMA,
        ],
    )
    def k(x_hbm, p_hbm, out_hbm, idx_v, rows_v, sem):
        wid = lax.axis_index("s") * NC + lax.axis_index("c")
        base = wid * n_per_w
        for c in range(n_ch):
            off = base + c * CH
            pltpu.sync_copy(p_hbm.at[pl.ds(off, CH)], idx_v)
            pltpu.sync_copy(x_hbm.at[pl.ds(off, CH)], rows_v)
            pltpu.async_copy(rows_v, out_hbm.at[idx_v], sem).wait()

    return k


CHH = 1024
NCH = H // CHH


def _mlp_body(ps_ref, nt_ref, x_ref, w12_ref, w3_ref, y_hbm,
              y_bufs, sem_out, cnt_ref):
    e = pl.program_id(0)
    ps = ps_ref[e]
    nt = nt_ref[e]

    @pl.when(e == 0)
    def _init():
        cnt_ref[0] = 0

    def tile(i, carry):
        r0 = pl.multiple_of(ps + i * TR, TR)
        xs = x_ref[pl.ds(r0, TR), :]
        acc = jnp.zeros((TR, D), jnp.float32)
        for c in range(NCH):
            w1c = w12_ref[0, c * CHH:(c + 1) * CHH, :]
            w2c = w12_ref[0, H + c * CHH:H + (c + 1) * CHH, :]
            h1 = lax.dot_general(xs, w1c, (((1,), (1,)), ((), ())),
                                 preferred_element_type=jnp.float32,
                                 precision=None)
            h2 = lax.dot_general(xs, w2c, (((1,), (1,)), ((), ())),
                                 preferred_element_type=jnp.float32,
                                 precision=None)
            g = (h1 / (1.0 + jnp.exp(-h1))) * h2
            w3c = w3_ref[0, :, c * CHH:(c + 1) * CHH]
            acc = acc + lax.dot_general(
                g, w3c, (((1,), (1,)), ((), ())),
                preferred_element_type=jnp.float32,
                precision=None)
        cnt = cnt_ref[0]
        k = lax.rem(cnt, 2)

        @pl.when(cnt >= 2)
        def _drain_prev():
            pltpu.make_async_copy(
                y_bufs.at[k], y_hbm.at[pl.ds(r0, TR), :], sem_out.at[k]
            ).wait()

        y_bufs[k] = acc
        pltpu.make_async_copy(
            y_bufs.at[k], y_hbm.at[pl.ds(r0, TR), :], sem_out.at[k]
        ).start()
        cnt_ref[0] = cnt + 1
        return carry

    lax.fori_loop(0, nt, tile, 0)

    @pl.when(e == E - 1)
    def _final_drain():
        cnt = cnt_ref[0]

        @pl.when(cnt >= 1)
        def _():
            k = lax.rem(cnt - 1, 2)
            pltpu.make_async_copy(
                y_bufs.at[k], y_hbm.at[pl.ds(0, TR), :], sem_out.at[k]
            ).wait()

        @pl.when(cnt >= 2)
        def _():
            k = lax.rem(cnt - 2, 2)
            pltpu.make_async_copy(
                y_bufs.at[k], y_hbm.at[pl.ds(0, TR), :], sem_out.at[k]
            ).wait()


def _mlp(pstarts, nts, x_pad, W12, W3):
    return pl.pallas_call(
        _mlp_body,
        grid=(E,),
        in_specs=[
            pl.BlockSpec(memory_space=pltpu.SMEM),
            pl.BlockSpec(memory_space=pltpu.SMEM),
            pl.BlockSpec((TP, D), lambda e: (0, 0)),
            pl.BlockSpec((1, 2 * H, D), lambda e: (e, 0, 0)),
            pl.BlockSpec((1, D, H), lambda e: (e, 0, 0)),
        ],
        out_specs=pl.BlockSpec(memory_space=pl.ANY),
        out_shape=jax.ShapeDtypeStruct((TP, D), jnp.float32),
        scratch_shapes=[
            pltpu.VMEM((2, TR, D), jnp.float32),
            pltpu.SemaphoreType.DMA((2,)),
            pltpu.SMEM((1,), jnp.int32),
        ],
    )(pstarts, nts, x_pad, W12, W3)


def kernel(x, Wr, W12, W3):
    B, S, Dm = x.shape
    x_flat = x.reshape(-1, Dm)

    p, nts, pstarts = _router(x_flat, Wr)

    x_pad = _make_sc_scatter(D, T, TP)(x_flat, p)
    y_pad = _mlp(pstarts, nts, x_pad, W12, W3)
    out = _make_sc_gather(D, T)(y_pad, p)
    return out.reshape(B, S, Dm)
